# Initial kernel scaffold; baseline (speedup 1.0000x reference)
import jax
import jax.numpy as jnp
from jax.experimental import pallas as pl


def kernel(mem_bank, query, recent_concept):
    raise NotImplementedError("write your pallas kernel here")


# speedup vs baseline: 1.2288x; 1.2288x over previous
import jax
import jax.numpy as jnp
from jax.experimental import pallas as pl
from jax.experimental.pallas import tpu as pltpu

_N = 1_000_000
_D = 64
_K = 8
_TAU = 0.07
_ALPHA = 0.8
_RB = 5000
_G = (_N // 2) // _RB
_NEG = -1e30
_BIG = 2**31 - 1


def _scan_body(blk_ref, qq_ref, vals_ref, idx_ref):
    b = pl.program_id(0)
    p = blk_ref[...] * qq_ref[...]
    se = jnp.sum(p[:, :_D], axis=1)
    so = jnp.sum(p[:, _D:], axis=1)
    s = jnp.concatenate([se[None, :], so[None, :]], axis=0)
    r = jax.lax.broadcasted_iota(jnp.int32, s.shape, 0)
    c = jax.lax.broadcasted_iota(jnp.int32, s.shape, 1)
    gidx = 2 * (b * _RB + c) + r
    lane = jax.lax.broadcasted_iota(jnp.int32, (1, 128), 1)
    vvec = jnp.full((1, 128), _NEG, jnp.float32)
    ivec = jnp.zeros((1, 128), jnp.int32)
    for k in range(_K):
        m = jnp.max(s)
        fk = jnp.min(jnp.where(s == m, gidx, _BIG))
        vvec = jnp.where(lane == k, m, vvec)
        ivec = jnp.where(lane == k, fk, ivec)
        s = jnp.where(gidx == fk, _NEG, s)
    vals_ref[...] = vvec.reshape(1, 1, 128)
    idx_ref[...] = ivec.reshape(1, 1, 128)


def _combine_body(vals_ref, idx_ref, rc_ref, bank_ref, out_ref, rows_v, sem):
    s = vals_ref[...].reshape(_G, 128)
    gi = idx_ref[...].reshape(_G, 128)
    tv, ti = [], []
    for k in range(_K):
        m = jnp.max(s)
        fk = jnp.min(jnp.where(s == m, gi, _BIG))
        tv.append(m)
        ti.append(fk)
        s = jnp.where(gi == fk, _NEG, s)
    cps = [
        pltpu.make_async_copy(
            bank_ref.at[pl.ds(ti[k], 1)], rows_v.at[pl.ds(k, 1)], sem)
        for k in range(_K)
    ]
    for cp in cps:
        cp.start()
    for cp in cps:
        cp.wait()
    m0 = tv[0]
    ws = [jnp.exp((tv[k] - m0) / _TAU) for k in range(_K)]
    den = ws[0]
    for k in range(1, _K):
        den = den + ws[k]
    ret = jnp.zeros((1, _D), jnp.float32)
    for k in range(_K):
        ret = ret + (ws[k] / den) * rows_v[k:k + 1, :]
    nrm = jnp.sqrt(jnp.sum(ret * ret))
    retn = ret / jnp.maximum(nrm, 1e-12)
    rc = rc_ref[...]
    scale = jnp.maximum(jnp.sqrt(jnp.sum(rc * rc)), 1e-6)
    out_ref[...] = _ALPHA * rc + (1.0 - _ALPHA) * retn * scale


def kernel(mem_bank, query, recent_concept):
    bank2 = mem_bank.reshape(_N // 2, 2 * _D)
    qq = jnp.concatenate([query, query]).reshape(1, 2 * _D)
    vals, idx = pl.pallas_call(
        _scan_body,
        grid=(_G,),
        in_specs=[
            pl.BlockSpec((_RB, 2 * _D), lambda b: (b, 0)),
            pl.BlockSpec((1, 2 * _D), lambda b: (0, 0)),
        ],
        out_specs=[
            pl.BlockSpec((1, 1, 128), lambda b: (b, 0, 0)),
            pl.BlockSpec((1, 1, 128), lambda b: (b, 0, 0)),
        ],
        out_shape=[
            jax.ShapeDtypeStruct((_G, 1, 128), jnp.float32),
            jax.ShapeDtypeStruct((_G, 1, 128), jnp.int32),
        ],
    )(bank2, qq)
    out = pl.pallas_call(
        _combine_body,
        in_specs=[
            pl.BlockSpec((_G, 1, 128), lambda: (0, 0, 0)),
            pl.BlockSpec((_G, 1, 128), lambda: (0, 0, 0)),
            pl.BlockSpec((1, _D), lambda: (0, 0)),
            pl.BlockSpec(memory_space=pl.ANY),
        ],
        out_specs=pl.BlockSpec((1, _D), lambda: (0, 0)),
        out_shape=jax.ShapeDtypeStruct((1, _D), jnp.float32),
        scratch_shapes=[
            pltpu.VMEM((_K, _D), jnp.float32),
            pltpu.SemaphoreType.DMA,
        ],
    )(vals, idx, recent_concept.reshape(1, _D), mem_bank)
    return out.reshape(_D)


# speedup vs baseline: 1.8300x; 1.4893x over previous
import jax
import jax.numpy as jnp
from jax.experimental import pallas as pl
from jax.experimental.pallas import tpu as pltpu

_N = 1_000_000
_D = 64
_K = 8
_TAU = 0.07
_ALPHA = 0.8
_Q = 4
_NQ = _N // _Q
_RB = 2000
_G = _NQ // _RB
_NEG = -1e30
_BIG = 2**31 - 1


def _scan_body(b0_ref, b1_ref, b2_ref, b3_ref, q_ref, vals_ref, idx_ref):
    b = pl.program_id(0)
    q = q_ref[...]
    parts = []
    for blk_ref in (b0_ref, b1_ref, b2_ref, b3_ref):
        p = blk_ref[...] * q
        parts.append(jnp.sum(p, axis=1)[None, :])
    s = jnp.concatenate(parts, axis=0)
    r = jax.lax.broadcasted_iota(jnp.int32, s.shape, 0)
    c = jax.lax.broadcasted_iota(jnp.int32, s.shape, 1)
    gidx = r * _NQ + b * _RB + c
    lane = jax.lax.broadcasted_iota(jnp.int32, (1, 128), 1)
    vvec = jnp.full((1, 128), _NEG, jnp.float32)
    ivec = jnp.zeros((1, 128), jnp.int32)
    for k in range(_K):
        m = jnp.max(s)
        fk = jnp.min(jnp.where(s == m, gidx, _BIG))
        vvec = jnp.where(lane == k, m, vvec)
        ivec = jnp.where(lane == k, fk, ivec)
        s = jnp.where(gidx == fk, _NEG, s)
    vals_ref[...] = vvec.reshape(1, 1, 128)
    idx_ref[...] = ivec.reshape(1, 1, 128)


def _combine_body(vals_ref, idx_ref, rc_ref, bank_ref, out_ref, rows_v, sem):
    s = vals_ref[...].reshape(_G, 128)
    gi = idx_ref[...].reshape(_G, 128)
    tv, ti = [], []
    for k in range(_K):
        m = jnp.max(s)
        fk = jnp.min(jnp.where(s == m, gi, _BIG))
        tv.append(m)
        ti.append(fk)
        s = jnp.where(gi == fk, _NEG, s)
    cps = [
        pltpu.make_async_copy(
            bank_ref.at[pl.ds(ti[k], 1)], rows_v.at[pl.ds(k, 1)], sem)
        for k in range(_K)
    ]
    for cp in cps:
        cp.start()
    for cp in cps:
        cp.wait()
    m0 = tv[0]
    ws = [jnp.exp((tv[k] - m0) / _TAU) for k in range(_K)]
    den = ws[0]
    for k in range(1, _K):
        den = den + ws[k]
    ret = jnp.zeros((1, _D), jnp.float32)
    for k in range(_K):
        ret = ret + (ws[k] / den) * rows_v[k:k + 1, :]
    nrm = jnp.sqrt(jnp.sum(ret * ret))
    retn = ret / jnp.maximum(nrm, 1e-12)
    rc = rc_ref[...]
    scale = jnp.maximum(jnp.sqrt(jnp.sum(rc * rc)), 1e-6)
    out_ref[...] = _ALPHA * rc + (1.0 - _ALPHA) * retn * scale


def _mk_slice_spec(qi):
    return pl.BlockSpec((_RB, _D), lambda b, _q=qi: (_q * _G + b, 0))


def kernel(mem_bank, query, recent_concept):
    vals, idx = pl.pallas_call(
        _scan_body,
        grid=(_G,),
        in_specs=[_mk_slice_spec(qi) for qi in range(_Q)] + [
            pl.BlockSpec((1, _D), lambda b: (0, 0)),
        ],
        out_specs=[
            pl.BlockSpec((1, 1, 128), lambda b: (b, 0, 0)),
            pl.BlockSpec((1, 1, 128), lambda b: (b, 0, 0)),
        ],
        out_shape=[
            jax.ShapeDtypeStruct((_G, 1, 128), jnp.float32),
            jax.ShapeDtypeStruct((_G, 1, 128), jnp.int32),
        ],
    )(mem_bank, mem_bank, mem_bank, mem_bank, query.reshape(1, _D))
    out = pl.pallas_call(
        _combine_body,
        in_specs=[
            pl.BlockSpec((_G, 1, 128), lambda: (0, 0, 0)),
            pl.BlockSpec((_G, 1, 128), lambda: (0, 0, 0)),
            pl.BlockSpec((1, _D), lambda: (0, 0)),
            pl.BlockSpec(memory_space=pl.ANY),
        ],
        out_specs=pl.BlockSpec((1, _D), lambda: (0, 0)),
        out_shape=jax.ShapeDtypeStruct((1, _D), jnp.float32),
        scratch_shapes=[
            pltpu.VMEM((_K, _D), jnp.float32),
            pltpu.SemaphoreType.DMA,
        ],
    )(vals, idx, recent_concept.reshape(1, _D), mem_bank)
    return out.reshape(_D)


# speedup vs baseline: 2.7422x; 1.4984x over previous
import jax
import jax.numpy as jnp
from jax.experimental import pallas as pl
from jax.experimental.pallas import tpu as pltpu

_N = 1_000_000
_D = 64
_K = 8
_TAU = 0.07
_ALPHA = 0.8
_Q = 8
_NQ = _N // _Q
_RB = 5000
_G = _NQ // _RB
_NEG = -1e30
_BIG = 2**31 - 1


def _scan_body(*refs):
    blk_refs = refs[:_Q]
    q_ref, vals_ref, idx_ref = refs[_Q], refs[_Q + 1], refs[_Q + 2]
    b = pl.program_id(0)
    q = q_ref[...]
    parts = []
    for blk_ref in blk_refs:
        p = blk_ref[...] * q
        parts.append(jnp.sum(p, axis=1)[None, :])
    s = jnp.concatenate(parts, axis=0)
    r = jax.lax.broadcasted_iota(jnp.int32, s.shape, 0)
    c = jax.lax.broadcasted_iota(jnp.int32, s.shape, 1)
    gidx = r * _NQ + b * _RB + c
    lane = jax.lax.broadcasted_iota(jnp.int32, (1, 128), 1)
    vvec = jnp.full((1, 128), _NEG, jnp.float32)
    ivec = jnp.zeros((1, 128), jnp.int32)
    for k in range(_K):
        m = jnp.max(s)
        fk = jnp.min(jnp.where(s == m, gidx, _BIG))
        vvec = jnp.where(lane == k, m, vvec)
        ivec = jnp.where(lane == k, fk, ivec)
        s = jnp.where(gidx == fk, _NEG, s)
    vals_ref[...] = vvec.reshape(1, 1, 128)
    idx_ref[...] = ivec.reshape(1, 1, 128)


def _combine_body(vals_ref, idx_ref, rc_ref, bank_ref, out_ref, rows_v, sem):
    s = vals_ref[...].reshape(_G, 128)
    gi = idx_ref[...].reshape(_G, 128)
    tv, ti = [], []
    for k in range(_K):
        m = jnp.max(s)
        fk = jnp.min(jnp.where(s == m, gi, _BIG))
        tv.append(m)
        ti.append(fk)
        s = jnp.where(gi == fk, _NEG, s)
    cps = [
        pltpu.make_async_copy(
            bank_ref.at[pl.ds(ti[k], 1)], rows_v.at[pl.ds(k, 1)], sem)
        for k in range(_K)
    ]
    for cp in cps:
        cp.start()
    for cp in cps:
        cp.wait()
    m0 = tv[0]
    ws = [jnp.exp((tv[k] - m0) / _TAU) for k in range(_K)]
    den = ws[0]
    for k in range(1, _K):
        den = den + ws[k]
    ret = jnp.zeros((1, _D), jnp.float32)
    for k in range(_K):
        ret = ret + (ws[k] / den) * rows_v[k:k + 1, :]
    nrm = jnp.sqrt(jnp.sum(ret * ret))
    retn = ret / jnp.maximum(nrm, 1e-12)
    rc = rc_ref[...]
    scale = jnp.maximum(jnp.sqrt(jnp.sum(rc * rc)), 1e-6)
    out_ref[...] = _ALPHA * rc + (1.0 - _ALPHA) * retn * scale


def _mk_slice_spec(qi):
    return pl.BlockSpec((_RB, _D), lambda b, _q=qi: (_q * _G + b, 0))


def kernel(mem_bank, query, recent_concept):
    vals, idx = pl.pallas_call(
        _scan_body,
        grid=(_G,),
        in_specs=[_mk_slice_spec(qi) for qi in range(_Q)] + [
            pl.BlockSpec((1, _D), lambda b: (0, 0)),
        ],
        out_specs=[
            pl.BlockSpec((1, 1, 128), lambda b: (b, 0, 0)),
            pl.BlockSpec((1, 1, 128), lambda b: (b, 0, 0)),
        ],
        out_shape=[
            jax.ShapeDtypeStruct((_G, 1, 128), jnp.float32),
            jax.ShapeDtypeStruct((_G, 1, 128), jnp.int32),
        ],
    )(*([mem_bank] * _Q), query.reshape(1, _D))
    out = pl.pallas_call(
        _combine_body,
        in_specs=[
            pl.BlockSpec((_G, 1, 128), lambda: (0, 0, 0)),
            pl.BlockSpec((_G, 1, 128), lambda: (0, 0, 0)),
            pl.BlockSpec((1, _D), lambda: (0, 0)),
            pl.BlockSpec(memory_space=pl.ANY),
        ],
        out_specs=pl.BlockSpec((1, _D), lambda: (0, 0)),
        out_shape=jax.ShapeDtypeStruct((1, _D), jnp.float32),
        scratch_shapes=[
            pltpu.VMEM((_K, _D), jnp.float32),
            pltpu.SemaphoreType.DMA,
        ],
    )(vals, idx, recent_concept.reshape(1, _D), mem_bank)
    return out.reshape(_D)


# speedup vs baseline: 14.7818x; 5.3906x over previous
import jax
import jax.numpy as jnp
from jax.experimental import pallas as pl
from jax.experimental.pallas import tpu as pltpu

_N = 1_000_000
_D = 64
_K = 8
_TAU = 0.07
_ALPHA = 0.8
_S = 8
_LC = 7808
_G = 16
_REG = _G * _LC
_MAIN = _S * _REG
_T = _N - _MAIN
_NEG = -1e30
_BIG = 2**31 - 1


def _scan_body(*refs):
    blk_refs = refs[:_S]
    tail_ref, q_ref, vals_ref, idx_ref = refs[_S], refs[_S + 1], refs[_S + 2], refs[_S + 3]
    b = pl.program_id(0)
    qv = q_ref[...]
    parts = []
    for blk_ref in blk_refs:
        p = blk_ref[...] * qv
        parts.append(jnp.sum(p, axis=0)[None, :])
    s8 = jnp.concatenate(parts, axis=0)
    st = jnp.sum(tail_ref[...] * qv, axis=0)[None, :]
    st = jnp.where(b == 0, st, _NEG)
    r = jax.lax.broadcasted_iota(jnp.int32, s8.shape, 0)
    c = jax.lax.broadcasted_iota(jnp.int32, s8.shape, 1)
    g8 = r * _REG + b * _LC + c
    gt = _MAIN + jax.lax.broadcasted_iota(jnp.int32, st.shape, 1)
    lane = jax.lax.broadcasted_iota(jnp.int32, (1, 128), 1)
    vvec = jnp.full((1, 128), _NEG, jnp.float32)
    ivec = jnp.zeros((1, 128), jnp.int32)
    for k in range(_K):
        m = jnp.maximum(jnp.max(s8), jnp.max(st))
        fk = jnp.minimum(
            jnp.min(jnp.where(s8 == m, g8, _BIG)),
            jnp.min(jnp.where(st == m, gt, _BIG)))
        vvec = jnp.where(lane == k, m, vvec)
        ivec = jnp.where(lane == k, fk, ivec)
        s8 = jnp.where(g8 == fk, _NEG, s8)
        st = jnp.where(gt == fk, _NEG, st)
    vals_ref[...] = vvec.reshape(1, 1, 128)
    idx_ref[...] = ivec.reshape(1, 1, 128)


def _combine_body(vals_ref, idx_ref, rc_ref, bank_ref, out_ref, cols_v, sem):
    s = vals_ref[...].reshape(_G, 128)
    gi = idx_ref[...].reshape(_G, 128)
    tv, ti = [], []
    for k in range(_K):
        m = jnp.max(s)
        fk = jnp.min(jnp.where(s == m, gi, _BIG))
        tv.append(m)
        ti.append(fk)
        s = jnp.where(gi == fk, _NEG, s)
    cps = [
        pltpu.make_async_copy(
            bank_ref.at[:, pl.ds((ti[k] // 128) * 128, 128)],
            cols_v.at[:, pl.ds(k * 128, 128)], sem)
        for k in range(_K)
    ]
    for cp in cps:
        cp.start()
    for cp in cps:
        cp.wait()
    m0 = tv[0]
    ws = [jnp.exp((tv[k] - m0) / _TAU) for k in range(_K)]
    den = ws[0]
    for k in range(1, _K):
        den = den + ws[k]
    lane64 = jax.lax.broadcasted_iota(jnp.int32, (_D, 128), 1)
    ret = jnp.zeros((_D, 1), jnp.float32)
    for k in range(_K):
        win = cols_v[:, k * 128:(k + 1) * 128]
        col = jnp.sum(
            jnp.where(lane64 == ti[k] % 128, win, 0.0), axis=1, keepdims=True)
        ret = ret + (ws[k] / den) * col
    nrm = jnp.sqrt(jnp.sum(ret * ret))
    retn = ret / jnp.maximum(nrm, 1e-12)
    rc = rc_ref[...]
    scale = jnp.maximum(jnp.sqrt(jnp.sum(rc * rc)), 1e-6)
    out_ref[...] = _ALPHA * rc + (1.0 - _ALPHA) * retn * scale


def _mk_stream_spec(si):
    return pl.BlockSpec((_D, _LC), lambda b, _s=si: (0, _s * _G + b))


def kernel(mem_bank, query, recent_concept):
    bank_t = mem_bank.T
    tail = jax.lax.slice(bank_t, (0, _MAIN), (_D, _N))
    vals, idx = pl.pallas_call(
        _scan_body,
        grid=(_G,),
        in_specs=[_mk_stream_spec(si) for si in range(_S)] + [
            pl.BlockSpec((_D, _T), lambda b: (0, 0)),
            pl.BlockSpec((_D, 1), lambda b: (0, 0)),
        ],
        out_specs=[
            pl.BlockSpec((1, 1, 128), lambda b: (b, 0, 0)),
            pl.BlockSpec((1, 1, 128), lambda b: (b, 0, 0)),
        ],
        out_shape=[
            jax.ShapeDtypeStruct((_G, 1, 128), jnp.float32),
            jax.ShapeDtypeStruct((_G, 1, 128), jnp.int32),
        ],
    )(*([bank_t] * _S), tail, query.reshape(_D, 1))
    out = pl.pallas_call(
        _combine_body,
        in_specs=[
            pl.BlockSpec((_G, 1, 128), lambda: (0, 0, 0)),
            pl.BlockSpec((_G, 1, 128), lambda: (0, 0, 0)),
            pl.BlockSpec((_D, 1), lambda: (0, 0)),
            pl.BlockSpec(memory_space=pl.ANY),
        ],
        out_specs=pl.BlockSpec((_D, 1), lambda: (0, 0)),
        out_shape=jax.ShapeDtypeStruct((_D, 1), jnp.float32),
        scratch_shapes=[
            pltpu.VMEM((_D, _K * 128), jnp.float32),
            pltpu.SemaphoreType.DMA,
        ],
    )(vals, idx, recent_concept.reshape(_D, 1), bank_t)
    return out.reshape(_D)
